# baseline (speedup 1.0000x reference)
import functools

import jax
import jax.numpy as jnp
from jax.experimental import pallas as pl
from jax.experimental.pallas import tpu as pltpu

_EPS = 1e-8
_TOP_K = 2
_ENTROPY_WEIGHT = 0.05
_MAX_USAGE_RATIO = 0.4


def _body(x_ref, gw_ref, gb_ref, ew_hbm, eb_ref, out_ref, aux_ref,
          wbuf, stat_ref, sems, *, n_tokens, n_experts):
    i = pl.program_id(0)
    n_tiles = pl.num_programs(0)

    @pl.when(i == 0)
    def _start_weight_copies():
        for e in range(n_experts):
            pltpu.make_async_copy(ew_hbm.at[e], wbuf.at[e], sems.at[e]).start()

    @pl.when(i == 0)
    def _init():
        stat_ref[...] = jnp.zeros_like(stat_ref)

    xb = x_ref[...]
    logits = jax.lax.dot_general(
        gw_ref[...], xb, (((1,), (1,)), ((), ())),
        preferred_element_type=jnp.float32) + gb_ref[...]
    m = jnp.max(logits, axis=0, keepdims=True)
    ex = jnp.exp(logits - m)
    probs = ex / jnp.sum(ex, axis=0, keepdims=True)
    ent_sum = -jnp.sum(probs * jnp.log(probs + _EPS))

    row = jax.lax.broadcasted_iota(jnp.int32, probs.shape, 0)
    p1 = jnp.max(probs, axis=0, keepdims=True)
    i1 = jnp.min(jnp.where(probs == p1, row, n_experts), axis=0,
                 keepdims=True)
    mask1 = row == i1
    probs_m = jnp.where(mask1, -jnp.inf, probs)
    p2 = jnp.max(probs_m, axis=0, keepdims=True)
    i2 = jnp.min(jnp.where(probs_m == p2, row, n_experts), axis=0,
                 keepdims=True)
    mask2 = row == i2
    combine_t = jnp.where(mask1, p1, 0.0) + jnp.where(mask2, p2, 0.0)

    counts_col = jnp.sum((mask1 | mask2).astype(jnp.float32), axis=1,
                         keepdims=True)
    lane128 = jax.lax.broadcasted_iota(jnp.int32, stat_ref.shape, 1)
    upd = jnp.where(lane128 == 0, counts_col, 0.0)
    upd = jnp.where(lane128 == 1, ent_sum / n_experts, upd)
    stat_ref[...] += upd

    combine = combine_t.T
    acc = jax.lax.dot_general(combine, eb_ref[...], (((1,), (0,)), ((), ())),
                              preferred_element_type=jnp.float32)
    half = n_experts // 2
    for e in range(n_experts):
        if e % 2 == 0:
            @pl.when(i == 0)
            def _wait_pair(e=e):
                for k in (e, e + 1):
                    pltpu.make_async_copy(
                        ew_hbm.at[k], wbuf.at[k], sems.at[k]).wait()
        y = jax.lax.dot_general(
            xb, wbuf[e], (((1,), (1,)), ((), ())),
            preferred_element_type=jnp.float32)
        acc = acc + combine[:, e:e + 1] * y
    out_ref[...] = acc

    @pl.when(i == n_tiles - 1)
    def _finish():
        tot = stat_ref[...]
        ent_total = jnp.sum(tot[:, 1:2])
        counts = tot[:, 0:1]
        usage = counts / (n_tokens + _EPS)
        penalty = jnp.sum(jnp.maximum(usage - _MAX_USAGE_RATIO, 0.0))
        aux = _ENTROPY_WEIGHT * ent_total / n_tokens + penalty
        aux_ref[...] = jnp.broadcast_to(aux, (1, 1))


def kernel(x, gate_w, gate_b, expert_w, expert_b):
    B, S, D = x.shape
    E, H, _ = expert_w.shape
    N = B * S
    T = 512
    x_flat = x.reshape(N, D)

    body = functools.partial(_body, n_tokens=N, n_experts=E)
    out, aux = pl.pallas_call(
        body,
        grid=(N // T,),
        in_specs=[
            pl.BlockSpec((T, D), lambda i: (i, 0)),
            pl.BlockSpec((E, D), lambda i: (0, 0)),
            pl.BlockSpec((E, 1), lambda i: (0, 0)),
            pl.BlockSpec(memory_space=pl.ANY),
            pl.BlockSpec((E, H), lambda i: (0, 0)),
        ],
        out_specs=[
            pl.BlockSpec((T, H), lambda i: (i, 0)),
            pl.BlockSpec((1, 1), lambda i: (0, 0)),
        ],
        out_shape=[
            jax.ShapeDtypeStruct((N, H), jnp.float32),
            jax.ShapeDtypeStruct((1, 1), jnp.float32),
        ],
        scratch_shapes=[
            pltpu.VMEM((E, H, D), jnp.float32),
            pltpu.VMEM((E, 128), jnp.float32),
            pltpu.SemaphoreType.DMA((E,)),
        ],
    )(x_flat, gate_w, gate_b.reshape(E, 1), expert_w, expert_b)
    return out.reshape(B, S, H), aux[0, 0]


# speedup vs baseline: 1.0601x; 1.0601x over previous
import functools

import jax
import jax.numpy as jnp
from jax.experimental import pallas as pl
from jax.experimental.pallas import tpu as pltpu

_EPS = 1e-8
_TOP_K = 2
_ENTROPY_WEIGHT = 0.05
_MAX_USAGE_RATIO = 0.4


def _body(x_ref, gw_ref, gb_ref, ew_hbm, eb_ref, out_ref, aux_ref,
          wbuf, stat_ref, sems, *, n_tokens, n_experts):
    i = pl.program_id(0)
    n_tiles = pl.num_programs(0)

    @pl.when(i == 0)
    def _start_weight_copies():
        for e in range(n_experts):
            pltpu.make_async_copy(ew_hbm.at[e], wbuf.at[e], sems.at[e]).start()

    @pl.when(i == 0)
    def _init():
        stat_ref[...] = jnp.zeros_like(stat_ref)

    xb = x_ref[...]
    logits = jax.lax.dot_general(
        gw_ref[...], xb, (((1,), (1,)), ((), ())),
        preferred_element_type=jnp.float32) + gb_ref[...]
    m = jnp.max(logits, axis=0, keepdims=True)
    ex = jnp.exp(logits - m)
    probs = ex / jnp.sum(ex, axis=0, keepdims=True)
    ent_sum = -jnp.sum(probs * jnp.log(probs + _EPS))

    row = jax.lax.broadcasted_iota(jnp.int32, probs.shape, 0)
    p1 = jnp.max(probs, axis=0, keepdims=True)
    i1 = jnp.min(jnp.where(probs == p1, row, n_experts), axis=0,
                 keepdims=True)
    mask1 = row == i1
    probs_m = jnp.where(mask1, -jnp.inf, probs)
    p2 = jnp.max(probs_m, axis=0, keepdims=True)
    i2 = jnp.min(jnp.where(probs_m == p2, row, n_experts), axis=0,
                 keepdims=True)
    mask2 = row == i2
    combine_t = jnp.where(mask1, p1, 0.0) + jnp.where(mask2, p2, 0.0)

    counts_col = jnp.sum((mask1 | mask2).astype(jnp.float32), axis=1,
                         keepdims=True)
    lane128 = jax.lax.broadcasted_iota(jnp.int32, stat_ref.shape, 1)
    upd = jnp.where(lane128 == 0, counts_col, 0.0)
    upd = jnp.where(lane128 == 1, ent_sum / n_experts, upd)
    stat_ref[...] += upd

    combine = combine_t.T
    acc = jax.lax.dot_general(combine, eb_ref[...], (((1,), (0,)), ((), ())),
                              preferred_element_type=jnp.float32)
    half = n_experts // 2
    for e in range(n_experts):
        if e % 2 == 0:
            @pl.when(i == 0)
            def _wait_pair(e=e):
                for k in (e, e + 1):
                    pltpu.make_async_copy(
                        ew_hbm.at[k], wbuf.at[k], sems.at[k]).wait()
        y = jax.lax.dot_general(
            xb, wbuf[e], (((1,), (1,)), ((), ())),
            preferred_element_type=jnp.float32)
        acc = acc + combine[:, e:e + 1] * y
    out_ref[...] = acc

    @pl.when(i == n_tiles - 1)
    def _finish():
        tot = stat_ref[...]
        ent_total = jnp.sum(tot[:, 1:2])
        counts = tot[:, 0:1]
        usage = counts / (n_tokens + _EPS)
        penalty = jnp.sum(jnp.maximum(usage - _MAX_USAGE_RATIO, 0.0))
        aux = _ENTROPY_WEIGHT * ent_total / n_tokens + penalty
        aux_ref[...] = jnp.broadcast_to(aux, (1, 1))


def kernel(x, gate_w, gate_b, expert_w, expert_b):
    B, S, D = x.shape
    E, H, _ = expert_w.shape
    N = B * S
    T = 2048
    x_flat = x.reshape(N, D)

    body = functools.partial(_body, n_tokens=N, n_experts=E)
    out, aux = pl.pallas_call(
        body,
        grid=(N // T,),
        in_specs=[
            pl.BlockSpec((T, D), lambda i: (i, 0)),
            pl.BlockSpec((E, D), lambda i: (0, 0)),
            pl.BlockSpec((E, 1), lambda i: (0, 0)),
            pl.BlockSpec(memory_space=pl.ANY),
            pl.BlockSpec((E, H), lambda i: (0, 0)),
        ],
        out_specs=[
            pl.BlockSpec((T, H), lambda i: (i, 0)),
            pl.BlockSpec((1, 1), lambda i: (0, 0)),
        ],
        out_shape=[
            jax.ShapeDtypeStruct((N, H), jnp.float32),
            jax.ShapeDtypeStruct((1, 1), jnp.float32),
        ],
        scratch_shapes=[
            pltpu.VMEM((E, H, D), jnp.float32),
            pltpu.VMEM((E, 128), jnp.float32),
            pltpu.SemaphoreType.DMA((E,)),
        ],
    )(x_flat, gate_w, gate_b.reshape(E, 1), expert_w, expert_b)
    return out.reshape(B, S, H), aux[0, 0]


# speedup vs baseline: 1.0965x; 1.0344x over previous
import functools

import jax
import jax.numpy as jnp
from jax.experimental import pallas as pl
from jax.experimental.pallas import tpu as pltpu

_EPS = 1e-8
_TOP_K = 2
_ENTROPY_WEIGHT = 0.05
_MAX_USAGE_RATIO = 0.4


def _body(x_ref, gw_ref, gb_ref, ew_hbm, eb_ref, out_ref, aux_ref,
          wbuf, stat_ref, sems, *, n_tokens, n_experts):
    i = pl.program_id(0)
    n_tiles = pl.num_programs(0)

    @pl.when(i == 0)
    def _start_weight_copies():
        for e in range(n_experts):
            pltpu.make_async_copy(ew_hbm.at[e], wbuf.at[e], sems.at[e]).start()

    @pl.when(i == 0)
    def _init():
        stat_ref[...] = jnp.zeros_like(stat_ref)

    xb = x_ref[...]
    logits = jax.lax.dot_general(
        gw_ref[...], xb, (((1,), (1,)), ((), ())),
        preferred_element_type=jnp.float32) + gb_ref[...]
    m = jnp.max(logits, axis=0, keepdims=True)
    ex = jnp.exp(logits - m)
    probs = ex / jnp.sum(ex, axis=0, keepdims=True)
    ent_sum = -jnp.sum(probs * jnp.log(probs + _EPS))

    row = jax.lax.broadcasted_iota(jnp.int32, probs.shape, 0)
    p1 = jnp.max(probs, axis=0, keepdims=True)
    i1 = jnp.min(jnp.where(probs == p1, row, n_experts), axis=0,
                 keepdims=True)
    mask1 = row == i1
    probs_m = jnp.where(mask1, -jnp.inf, probs)
    p2 = jnp.max(probs_m, axis=0, keepdims=True)
    i2 = jnp.min(jnp.where(probs_m == p2, row, n_experts), axis=0,
                 keepdims=True)
    mask2 = row == i2
    combine_t = jnp.where(mask1, p1, 0.0) + jnp.where(mask2, p2, 0.0)

    counts_col = jnp.sum((mask1 | mask2).astype(jnp.float32), axis=1,
                         keepdims=True)
    lane128 = jax.lax.broadcasted_iota(jnp.int32, stat_ref.shape, 1)
    upd = jnp.where(lane128 == 0, counts_col, 0.0)
    upd = jnp.where(lane128 == 1, ent_sum / n_experts, upd)
    stat_ref[...] += upd

    combine = combine_t.T
    acc = jax.lax.dot_general(combine, eb_ref[...], (((1,), (0,)), ((), ())),
                              preferred_element_type=jnp.float32)
    half = n_experts // 2
    for e in range(n_experts):
        if e % 2 == 0:
            @pl.when(i == 0)
            def _wait_pair(e=e):
                for k in (e, e + 1):
                    pltpu.make_async_copy(
                        ew_hbm.at[k], wbuf.at[k], sems.at[k]).wait()
        y = jax.lax.dot_general(
            xb, wbuf[e], (((1,), (1,)), ((), ())),
            preferred_element_type=jnp.float32)
        acc = acc + combine[:, e:e + 1] * y
    out_ref[...] = acc

    @pl.when(i == n_tiles - 1)
    def _finish():
        tot = stat_ref[...]
        ent_total = jnp.sum(tot[:, 1:2])
        counts = tot[:, 0:1]
        usage = counts / (n_tokens + _EPS)
        penalty = jnp.sum(jnp.maximum(usage - _MAX_USAGE_RATIO, 0.0))
        aux = _ENTROPY_WEIGHT * ent_total / n_tokens + penalty
        aux_ref[...] = jnp.broadcast_to(aux, (1, 1))


def kernel(x, gate_w, gate_b, expert_w, expert_b):
    B, S, D = x.shape
    E, H, _ = expert_w.shape
    N = B * S
    T = 1024
    x_flat = x.reshape(N, D)

    body = functools.partial(_body, n_tokens=N, n_experts=E)
    out, aux = pl.pallas_call(
        body,
        grid=(N // T,),
        in_specs=[
            pl.BlockSpec((T, D), lambda i: (i, 0)),
            pl.BlockSpec((E, D), lambda i: (0, 0)),
            pl.BlockSpec((E, 1), lambda i: (0, 0)),
            pl.BlockSpec(memory_space=pl.ANY),
            pl.BlockSpec((E, H), lambda i: (0, 0)),
        ],
        out_specs=[
            pl.BlockSpec((T, H), lambda i: (i, 0)),
            pl.BlockSpec((1, 1), lambda i: (0, 0)),
        ],
        out_shape=[
            jax.ShapeDtypeStruct((N, H), jnp.float32),
            jax.ShapeDtypeStruct((1, 1), jnp.float32),
        ],
        scratch_shapes=[
            pltpu.VMEM((E, H, D), jnp.float32),
            pltpu.VMEM((E, 128), jnp.float32),
            pltpu.SemaphoreType.DMA((E,)),
        ],
    )(x_flat, gate_w, gate_b.reshape(E, 1), expert_w, expert_b)
    return out.reshape(B, S, H), aux[0, 0]
